# Initial kernel scaffold; baseline (speedup 1.0000x reference)
import jax
import jax.numpy as jnp
from jax.experimental import pallas as pl


def kernel(x, edge_index, W1, b1, W2, b2):
    raise NotImplementedError("write your pallas kernel here")


# speedup vs baseline: 25.9092x; 25.9092x over previous
import functools

import jax
import jax.numpy as jnp
from jax import lax
from jax.experimental import pallas as pl
from jax.experimental.pallas import tpu as pltpu
from jax.experimental.pallas import tpu_sc as plsc

NC = 2
NS = 16
NW = NC * NS
CHUNK = 128


def _mesh():
  return plsc.VectorSubcoreMesh(
      core_axis_name="c", subcore_axis_name="s", num_cores=NC,
      num_subcores=NS)


def _hist_kernel(nacc, nch):
  rpt = nacc // NS

  @functools.partial(
      pl.kernel,
      out_type=jax.ShapeDtypeStruct((NC * nacc,), jnp.float32),
      mesh=_mesh(),
      scratch_types=[
          pltpu.VMEM((nch, CHUNK), jnp.int32),
          pltpu.VMEM((CHUNK,), jnp.float32),
          pltpu.VMEM((rpt,), jnp.float32),
          pltpu.VMEM_SHARED((nacc,), jnp.float32),
      ],
  )
  def k(rows_hbm, zeros_hbm, out_hbm, ridx_v, ones_v, zbuf_v, acc):
    c = lax.axis_index("c")
    s = lax.axis_index("s")
    wid = c * NS + s
    for i in range(CHUNK // 16):
      ones_v[pl.ds(16 * i, 16)] = jnp.ones((16,), jnp.float32)
    pltpu.sync_copy(zeros_hbm.at[pl.ds(s * rpt, rpt)], zbuf_v)
    pltpu.sync_copy(zbuf_v, acc.at[pl.ds(s * rpt, rpt)])
    pltpu.sync_copy(rows_hbm.at[wid], ridx_v)
    plsc.subcore_barrier()

    def body(j, carry):
      pltpu.sync_copy(ones_v, acc.at[ridx_v.at[j]], add=True)
      return carry

    lax.fori_loop(0, nch, body, 0)
    plsc.subcore_barrier()
    pltpu.sync_copy(acc.at[pl.ds(s * rpt, rpt)], zbuf_v)
    pltpu.sync_copy(zbuf_v, out_hbm.at[pl.ds(c * nacc + s * rpt, rpt)])

  return k


def _msg_kernel(nacc, nch, f):
  rpt = nacc // NS

  @functools.partial(
      pl.kernel,
      out_type=jax.ShapeDtypeStruct((NC, nacc, f), jnp.float32),
      mesh=_mesh(),
      scratch_types=[
          pltpu.VMEM((nch, CHUNK), jnp.int32),
          pltpu.VMEM((nch, CHUNK), jnp.int32),
          pltpu.VMEM((CHUNK, f), jnp.float32),
          pltpu.VMEM((rpt, f), jnp.float32),
          pltpu.VMEM_SHARED((nacc, f), jnp.float32),
          pltpu.SemaphoreType.DMA,
      ],
      compiler_params=pltpu.CompilerParams(use_tc_tiling_on_sc=False),
  )
  def k(rows_hbm, cols_hbm, g_hbm, zeros_hbm, out_hbm,
        ridx_v, cidx_v, msg_v, zbuf_v, acc, sem):
    c = lax.axis_index("c")
    s = lax.axis_index("s")
    wid = c * NS + s
    pltpu.sync_copy(zeros_hbm.at[pl.ds(s * rpt, rpt)], zbuf_v)
    pltpu.sync_copy(zbuf_v, acc.at[pl.ds(s * rpt, rpt)])
    pltpu.sync_copy(rows_hbm.at[wid], ridx_v)
    pltpu.sync_copy(cols_hbm.at[wid], cidx_v)
    plsc.subcore_barrier()

    def body(j, carry):
      pltpu.async_copy(g_hbm.at[cidx_v.at[j]], msg_v, sem).wait()
      pltpu.sync_copy(msg_v, acc.at[ridx_v.at[j]], add=True)
      return carry

    lax.fori_loop(0, nch, body, 0)
    plsc.subcore_barrier()
    pltpu.sync_copy(acc.at[pl.ds(s * rpt, rpt)], zbuf_v)
    pltpu.sync_copy(zbuf_v, out_hbm.at[c, pl.ds(s * rpt, rpt)])

  return k


def _mm_body(x_ref, w_ref, o_ref):
  o_ref[...] = jnp.dot(x_ref[...], w_ref[...],
                       preferred_element_type=jnp.float32)


def _scale_body(hist_ref, h_ref, o_ref):
  deg = 1.0 + hist_ref[:, 0] + hist_ref[:, 1]
  dinv = lax.rsqrt(deg)
  o_ref[...] = h_ref[...] * dinv[:, None]


def _layer1_body(hist_ref, s0_ref, s1_ref, g_ref, b_ref, w_ref, o_ref):
  deg = 1.0 + hist_ref[:, 0] + hist_ref[:, 1]
  dinv = lax.rsqrt(deg)
  pre = (s0_ref[...] + s1_ref[...] + g_ref[...]) * dinv[:, None] + b_ref[...]
  h1 = jnp.maximum(pre, 0.0)
  h2 = jnp.dot(h1, w_ref[...], preferred_element_type=jnp.float32)
  o_ref[...] = h2 * dinv[:, None]


def _layer2_body(hist_ref, s0_ref, s1_ref, g_ref, b_ref, o_ref):
  deg = 1.0 + hist_ref[:, 0] + hist_ref[:, 1]
  dinv = lax.rsqrt(deg)
  o = (s0_ref[...] + s1_ref[...] + g_ref[...]) * dinv[:, None] + b_ref[...]
  m = jnp.max(o, axis=1, keepdims=True)
  lse = jnp.log(jnp.sum(jnp.exp(o - m), axis=1, keepdims=True)) + m
  o_ref[...] = o - lse


def _row_call(body, nrows, blk, out_width, in_specs, out_dtype=jnp.float32):
  return pl.pallas_call(
      body,
      grid=(nrows // blk,),
      in_specs=in_specs,
      out_specs=pl.BlockSpec((blk, out_width), lambda i: (i, 0)),
      out_shape=jax.ShapeDtypeStruct((nrows, out_width), out_dtype),
  )


def kernel(x, edge_index, W1, b1, W2, b2):
  n, d = x.shape
  h = W1.shape[1]
  cdim = W2.shape[1]
  e = edge_index.shape[1]

  ew = -(-e // (NW * CHUNK)) * CHUNK
  epad = ew * NW
  nch = ew // CHUNK
  pad = epad - e
  rows = jnp.concatenate(
      [edge_index[0], jnp.full((pad,), n, jnp.int32)]).reshape(NW, nch, CHUNK)
  cols = jnp.concatenate(
      [edge_index[1], jnp.zeros((pad,), jnp.int32)]).reshape(NW, nch, CHUNK)

  nacc = -(-(n + 1) // 256) * 256
  blk = nacc // 16

  zh = jnp.zeros((nacc,), jnp.float32)
  z1 = jnp.zeros((nacc, h), jnp.float32)
  z2 = jnp.zeros((nacc, cdim), jnp.float32)
  x_pad = jnp.concatenate([x, jnp.zeros((nacc - n, d), x.dtype)])

  hist = _hist_kernel(nacc, nch)(rows, zh)
  hist_t = hist.reshape(NC, nacc).T

  hmat = _row_call(
      _mm_body, nacc, blk, h,
      [pl.BlockSpec((blk, d), lambda i: (i, 0)),
       pl.BlockSpec((d, h), lambda i: (0, 0))])(x_pad, W1)
  g1 = _row_call(
      _scale_body, nacc, blk, h,
      [pl.BlockSpec((blk, NC), lambda i: (i, 0)),
       pl.BlockSpec((blk, h), lambda i: (i, 0))])(hist_t, hmat)

  s1 = _msg_kernel(nacc, nch, h)(rows, cols, g1, z1)

  g2 = _row_call(
      _layer1_body, nacc, blk, cdim,
      [pl.BlockSpec((blk, NC), lambda i: (i, 0)),
       pl.BlockSpec((blk, h), lambda i: (i, 0)),
       pl.BlockSpec((blk, h), lambda i: (i, 0)),
       pl.BlockSpec((blk, h), lambda i: (i, 0)),
       pl.BlockSpec((1, h), lambda i: (0, 0)),
       pl.BlockSpec((h, cdim), lambda i: (0, 0))])(
           hist_t, s1[0], s1[1], g1, b1[None, :], W2)

  s2 = _msg_kernel(nacc, nch, cdim)(rows, cols, g2, z2)

  out = _row_call(
      _layer2_body, nacc, blk, cdim,
      [pl.BlockSpec((blk, NC), lambda i: (i, 0)),
       pl.BlockSpec((blk, cdim), lambda i: (i, 0)),
       pl.BlockSpec((blk, cdim), lambda i: (i, 0)),
       pl.BlockSpec((blk, cdim), lambda i: (i, 0)),
       pl.BlockSpec((1, cdim), lambda i: (0, 0))])(
           hist_t, s2[0], s2[1], g2, b2[None, :])

  return out[:n]
